# baseline (device time: 426797 ns/iter reference)
import jax
import jax.numpy as jnp
from jax import lax
from jax.experimental import pallas as pl
from jax.experimental.pallas import tpu as pltpu

N_DEV = 16
M = 2048
N = 2048
CHUNK = M // N_DEV


def kernel(A, B):
    partial = jnp.dot(
        A.astype(jnp.bfloat16),
        B.astype(jnp.bfloat16),
        preferred_element_type=jnp.float32,
    )

    def body(p_ref, out_ref, recv_ref, rs_send_sems, rs_recv_sems,
             ag_send_sems, ag_recv_sems):
        my = lax.axis_index("i")
        left = lax.rem(my - 1 + N_DEV, N_DEV)
        right = lax.rem(my + 1, N_DEV)

        barrier_sem = pltpu.get_barrier_semaphore()
        for nbr in (left, right):
            pl.semaphore_signal(
                barrier_sem, inc=1,
                device_id=(nbr,), device_id_type=pl.DeviceIdType.MESH,
            )
        pl.semaphore_wait(barrier_sem, 2)

        out_ref[...] = p_ref[...]

        for h in range(N_DEV - 1):
            s_idx = lax.rem(my - h + 2 * N_DEV, N_DEV)
            r_idx = lax.rem(my - h - 1 + 2 * N_DEV, N_DEV)
            rdma = pltpu.make_async_remote_copy(
                src_ref=out_ref.at[pl.ds(s_idx * CHUNK, CHUNK), :],
                dst_ref=recv_ref.at[h],
                send_sem=rs_send_sems.at[h],
                recv_sem=rs_recv_sems.at[h],
                device_id=(right,),
                device_id_type=pl.DeviceIdType.MESH,
            )
            rdma.start()
            rdma.wait()
            out_ref[pl.ds(r_idx * CHUNK, CHUNK), :] += recv_ref[h]

        for h in range(N_DEV - 1):
            s_idx = lax.rem(my + 1 - h + 2 * N_DEV, N_DEV)
            rdma = pltpu.make_async_remote_copy(
                src_ref=out_ref.at[pl.ds(s_idx * CHUNK, CHUNK), :],
                dst_ref=out_ref.at[pl.ds(s_idx * CHUNK, CHUNK), :],
                send_sem=ag_send_sems.at[h],
                recv_sem=ag_recv_sems.at[h],
                device_id=(right,),
                device_id_type=pl.DeviceIdType.MESH,
            )
            rdma.start()
            rdma.wait()

    return pl.pallas_call(
        body,
        out_shape=jax.ShapeDtypeStruct((M, N), jnp.float32),
        in_specs=[pl.BlockSpec(memory_space=pltpu.VMEM)],
        out_specs=pl.BlockSpec(memory_space=pltpu.VMEM),
        scratch_shapes=[
            pltpu.VMEM((N_DEV - 1, CHUNK, N), jnp.float32),
            pltpu.SemaphoreType.DMA((N_DEV - 1,)),
            pltpu.SemaphoreType.DMA((N_DEV - 1,)),
            pltpu.SemaphoreType.DMA((N_DEV - 1,)),
            pltpu.SemaphoreType.DMA((N_DEV - 1,)),
        ],
        compiler_params=pltpu.CompilerParams(collective_id=0),
    )(partial)


# device time: 259872 ns/iter; 1.6423x vs baseline; 1.6423x over previous
import jax
import jax.numpy as jnp
from jax import lax
from jax.experimental import pallas as pl
from jax.experimental.pallas import tpu as pltpu

N_DEV = 16
M = 2048
N = 2048
CHUNK = M // N_DEV


def kernel(A, B):
    partial = jnp.dot(
        A.astype(jnp.bfloat16),
        B.astype(jnp.bfloat16),
        preferred_element_type=jnp.float32,
    )

    def body(p_ref, out_ref, rs_send, ag_buf,
             rs_send_sems, rs_recv_sems, ag_send_sems, ag_recv_sems):
        my = lax.axis_index("i")
        left = lax.rem(my - 1 + N_DEV, N_DEV)
        right = lax.rem(my + 1, N_DEV)

        barrier_sem = pltpu.get_barrier_semaphore()
        for nbr in (left, right):
            pl.semaphore_signal(
                barrier_sem, inc=1,
                device_id=(nbr,), device_id_type=pl.DeviceIdType.MESH,
            )
        pl.semaphore_wait(barrier_sem, 2)

        out_ref[...] = p_ref[...]

        rs_rdmas = []
        for h in range(N_DEV - 1):
            s_idx = lax.rem(my - h + 2 * N_DEV, N_DEV)
            r_idx = lax.rem(my - h - 1 + 2 * N_DEV, N_DEV)
            slot = h % 2
            if h >= 2:
                rs_rdmas[h - 2].wait_send()
            rs_send[slot] = out_ref[pl.ds(s_idx * CHUNK, CHUNK), :].astype(
                jnp.bfloat16)
            rdma = pltpu.make_async_remote_copy(
                src_ref=rs_send.at[slot],
                dst_ref=ag_buf.at[s_idx],
                send_sem=rs_send_sems.at[h],
                recv_sem=rs_recv_sems.at[h],
                device_id=(right,),
                device_id_type=pl.DeviceIdType.MESH,
            )
            rdma.start()
            rs_rdmas.append(rdma)
            rdma.wait_recv()
            out_ref[pl.ds(r_idx * CHUNK, CHUNK), :] += ag_buf[r_idx].astype(
                jnp.float32)
        rs_rdmas[N_DEV - 3].wait_send()
        rs_rdmas[N_DEV - 2].wait_send()

        own = lax.rem(my + 1, N_DEV)
        ag_buf[own] = out_ref[pl.ds(own * CHUNK, CHUNK), :].astype(jnp.bfloat16)

        ag_rdmas = []
        for h in range(N_DEV - 1):
            s_idx = lax.rem(my + 1 - h + 2 * N_DEV, N_DEV)
            r_idx = lax.rem(my - h + 2 * N_DEV, N_DEV)
            rdma = pltpu.make_async_remote_copy(
                src_ref=ag_buf.at[s_idx],
                dst_ref=ag_buf.at[s_idx],
                send_sem=ag_send_sems.at[h],
                recv_sem=ag_recv_sems.at[h],
                device_id=(right,),
                device_id_type=pl.DeviceIdType.MESH,
            )
            rdma.start()
            ag_rdmas.append(rdma)
            rdma.wait_recv()
            out_ref[pl.ds(r_idx * CHUNK, CHUNK), :] = ag_buf[r_idx].astype(
                jnp.float32)
        for rdma in ag_rdmas:
            rdma.wait_send()

    return pl.pallas_call(
        body,
        out_shape=jax.ShapeDtypeStruct((M, N), jnp.float32),
        in_specs=[pl.BlockSpec(memory_space=pltpu.VMEM)],
        out_specs=pl.BlockSpec(memory_space=pltpu.VMEM),
        scratch_shapes=[
            pltpu.VMEM((2, CHUNK, N), jnp.bfloat16),
            pltpu.VMEM((N_DEV, CHUNK, N), jnp.bfloat16),
            pltpu.SemaphoreType.DMA((N_DEV - 1,)),
            pltpu.SemaphoreType.DMA((N_DEV - 1,)),
            pltpu.SemaphoreType.DMA((N_DEV - 1,)),
            pltpu.SemaphoreType.DMA((N_DEV - 1,)),
        ],
        compiler_params=pltpu.CompilerParams(collective_id=0),
    )(partial)


# device time: 215234 ns/iter; 1.9829x vs baseline; 1.2074x over previous
import jax
import jax.numpy as jnp
from jax import lax
from jax.experimental import pallas as pl
from jax.experimental.pallas import tpu as pltpu

N_DEV = 16
M = 2048
N = 2048
CHUNK = M // N_DEV
HALF = N // 2


def kernel(A, B):
    partial = jnp.dot(
        A.astype(jnp.bfloat16),
        B.astype(jnp.bfloat16),
        preferred_element_type=jnp.float32,
    )

    def body(p_ref, out_ref, stage_r, stage_l, ag_r, ag_l,
             rs_s_r, rs_r_r, rs_s_l, rs_r_l,
             ag_s_r, ag_r_r, ag_s_l, ag_r_l):
        my = lax.axis_index("i")
        left = lax.rem(my - 1 + N_DEV, N_DEV)
        right = lax.rem(my + 1, N_DEV)

        barrier_sem = pltpu.get_barrier_semaphore()
        for nbr in (left, right):
            pl.semaphore_signal(
                barrier_sem, inc=1,
                device_id=(nbr,), device_id_type=pl.DeviceIdType.MESH,
            )
        pl.semaphore_wait(barrier_sem, 2)

        out_ref[...] = p_ref[...]

        dirs = [
            dict(sgn=-1, nbr=right, col=0, stage=stage_r, ag=ag_r,
                 rs_s=rs_s_r, rs_r=rs_r_r, ag_s=ag_s_r, ag_r=ag_r_r),
            dict(sgn=+1, nbr=left, col=HALF, stage=stage_l, ag=ag_l,
                 rs_s=rs_s_l, rs_r=rs_r_l, ag_s=ag_s_l, ag_r=ag_r_l),
        ]

        def idx(k):
            return lax.rem(k + 4 * N_DEV, N_DEV)

        rs_rdmas = {-1: [], +1: []}
        for h in range(N_DEV - 1):
            slot = h % 2
            for d in dirs:
                s_idx = idx(my + d["sgn"] * h)
                if h >= 2:
                    rs_rdmas[d["sgn"]][h - 2].wait_send()
                d["stage"][slot] = out_ref[
                    pl.ds(s_idx * CHUNK, CHUNK), d["col"]:d["col"] + HALF
                ].astype(jnp.bfloat16)
                rdma = pltpu.make_async_remote_copy(
                    src_ref=d["stage"].at[slot],
                    dst_ref=d["ag"].at[s_idx],
                    send_sem=d["rs_s"].at[h],
                    recv_sem=d["rs_r"].at[h],
                    device_id=(d["nbr"],),
                    device_id_type=pl.DeviceIdType.MESH,
                )
                rdma.start()
                rs_rdmas[d["sgn"]].append(rdma)
            for d in dirs:
                r_idx = idx(my + d["sgn"] * (h + 1))
                rs_rdmas[d["sgn"]][h].wait_recv()
                out_ref[
                    pl.ds(r_idx * CHUNK, CHUNK), d["col"]:d["col"] + HALF
                ] += d["ag"][r_idx].astype(jnp.float32)
        for sgn in (-1, +1):
            rs_rdmas[sgn][N_DEV - 3].wait_send()
            rs_rdmas[sgn][N_DEV - 2].wait_send()

        for d in dirs:
            own = idx(my - d["sgn"])
            d["ag"][own] = out_ref[
                pl.ds(own * CHUNK, CHUNK), d["col"]:d["col"] + HALF
            ].astype(jnp.bfloat16)

        ag_rdmas = {-1: [], +1: []}
        for h in range(N_DEV - 1):
            for d in dirs:
                s_idx = idx(my - d["sgn"] + d["sgn"] * h)
                rdma = pltpu.make_async_remote_copy(
                    src_ref=d["ag"].at[s_idx],
                    dst_ref=d["ag"].at[s_idx],
                    send_sem=d["ag_s"].at[h],
                    recv_sem=d["ag_r"].at[h],
                    device_id=(d["nbr"],),
                    device_id_type=pl.DeviceIdType.MESH,
                )
                rdma.start()
                ag_rdmas[d["sgn"]].append(rdma)
            for d in dirs:
                r_idx = idx(my + d["sgn"] * h)
                ag_rdmas[d["sgn"]][h].wait_recv()
                out_ref[
                    pl.ds(r_idx * CHUNK, CHUNK), d["col"]:d["col"] + HALF
                ] = d["ag"][r_idx].astype(jnp.float32)
        for sgn in (-1, +1):
            for rdma in ag_rdmas[sgn]:
                rdma.wait_send()

    return pl.pallas_call(
        body,
        out_shape=jax.ShapeDtypeStruct((M, N), jnp.float32),
        in_specs=[pl.BlockSpec(memory_space=pltpu.VMEM)],
        out_specs=pl.BlockSpec(memory_space=pltpu.VMEM),
        scratch_shapes=[
            pltpu.VMEM((2, CHUNK, HALF), jnp.bfloat16),
            pltpu.VMEM((2, CHUNK, HALF), jnp.bfloat16),
            pltpu.VMEM((N_DEV, CHUNK, HALF), jnp.bfloat16),
            pltpu.VMEM((N_DEV, CHUNK, HALF), jnp.bfloat16),
            pltpu.SemaphoreType.DMA((N_DEV - 1,)),
            pltpu.SemaphoreType.DMA((N_DEV - 1,)),
            pltpu.SemaphoreType.DMA((N_DEV - 1,)),
            pltpu.SemaphoreType.DMA((N_DEV - 1,)),
            pltpu.SemaphoreType.DMA((N_DEV - 1,)),
            pltpu.SemaphoreType.DMA((N_DEV - 1,)),
            pltpu.SemaphoreType.DMA((N_DEV - 1,)),
            pltpu.SemaphoreType.DMA((N_DEV - 1,)),
        ],
        compiler_params=pltpu.CompilerParams(collective_id=0),
    )(partial)


# device time: 147425 ns/iter; 2.8950x vs baseline; 1.4600x over previous
import jax
import jax.numpy as jnp
from jax import lax
from jax.experimental import pallas as pl
from jax.experimental.pallas import tpu as pltpu

N_DEV = 16
M = 2048
N = 2048
CHUNK = M // N_DEV
HALF = N // 2
SUB = 2
QCOL = HALF // SUB


def kernel(A, B):
    partial = jnp.dot(
        A.astype(jnp.bfloat16),
        B.astype(jnp.bfloat16),
        preferred_element_type=jnp.float32,
    )

    def body(p_ref, out_ref, stage_r, stage_l, ag_r, ag_l,
             send_sems_r, recv_sems_r, send_sems_l, recv_sems_l):
        my = lax.axis_index("i")
        left = lax.rem(my - 1 + N_DEV, N_DEV)
        right = lax.rem(my + 1, N_DEV)

        barrier_sem = pltpu.get_barrier_semaphore()
        for nbr in (left, right):
            pl.semaphore_signal(
                barrier_sem, inc=1,
                device_id=(nbr,), device_id_type=pl.DeviceIdType.MESH,
            )
        pl.semaphore_wait(barrier_sem, 2)

        dirs = [
            dict(sgn=-1, nbr=right, col=0, stage=stage_r, ag=ag_r,
                 ssem=send_sems_r, rsem=recv_sems_r),
            dict(sgn=+1, nbr=left, col=HALF, stage=stage_l, ag=ag_l,
                 ssem=send_sems_l, rsem=recv_sems_l),
        ]

        def idx(k):
            return lax.rem(k + 4 * N_DEV, N_DEV)

        def send(d, j, h, src_buf, src_slice):
            jsl = slice(j * QCOL, (j + 1) * QCOL)
            rdma = pltpu.make_async_remote_copy(
                src_ref=src_buf.at[src_slice, :, jsl],
                dst_ref=d["ag"].at[idx(my + d["sgn"] * h), :, jsl],
                send_sem=d["ssem"].at[h, j],
                recv_sem=d["rsem"].at[h, j],
                device_id=(d["nbr"],),
                device_id_type=pl.DeviceIdType.MESH,
            )
            rdma.start()
            return rdma

        rs = {}
        for d in dirs:
            s0 = idx(my)
            d["stage"][0] = p_ref[
                pl.ds(s0 * CHUNK, CHUNK), d["col"]:d["col"] + HALF
            ].astype(jnp.bfloat16)
            for j in range(SUB):
                rs[(d["sgn"], j)] = [send(d, j, 0, d["stage"], 0)]

        for h in range(N_DEV - 1):
            for j in range(SUB):
                for d in dirs:
                    cj = d["col"] + j * QCOL
                    r_idx = idx(my + d["sgn"] * (h + 1))
                    rd = rs[(d["sgn"], j)]
                    rd[h].wait_recv()
                    out_ref[pl.ds(r_idx * CHUNK, CHUNK), cj:cj + QCOL] = (
                        p_ref[pl.ds(r_idx * CHUNK, CHUNK), cj:cj + QCOL]
                        + d["ag"][r_idx, :, j * QCOL:(j + 1) * QCOL].astype(
                            jnp.float32)
                    )
                    if h < N_DEV - 2:
                        slot = (h + 1) % 2
                        if h >= 1:
                            rd[h - 1].wait_send()
                        d["stage"][slot, :, j * QCOL:(j + 1) * QCOL] = out_ref[
                            pl.ds(r_idx * CHUNK, CHUNK), cj:cj + QCOL
                        ].astype(jnp.bfloat16)
                        rd.append(send(d, j, h + 1, d["stage"], slot))
        for key, rd in rs.items():
            rd[N_DEV - 3].wait_send()
            rd[N_DEV - 2].wait_send()

        def ag_send(d, j, h):
            jsl = slice(j * QCOL, (j + 1) * QCOL)
            s_idx = idx(my - d["sgn"] + d["sgn"] * h)
            rdma = pltpu.make_async_remote_copy(
                src_ref=d["ag"].at[s_idx, :, jsl],
                dst_ref=d["ag"].at[s_idx, :, jsl],
                send_sem=d["ssem"].at[h, j],
                recv_sem=d["rsem"].at[h, j],
                device_id=(d["nbr"],),
                device_id_type=pl.DeviceIdType.MESH,
            )
            rdma.start()
            return rdma

        ag = {}
        for d in dirs:
            own = idx(my - d["sgn"])
            d["ag"][own] = out_ref[
                pl.ds(own * CHUNK, CHUNK), d["col"]:d["col"] + HALF
            ].astype(jnp.bfloat16)
            for j in range(SUB):
                ag[(d["sgn"], j)] = [ag_send(d, j, 0)]

        for h in range(N_DEV - 1):
            for j in range(SUB):
                for d in dirs:
                    cj = d["col"] + j * QCOL
                    r_idx = idx(my + d["sgn"] * h)
                    ag[(d["sgn"], j)][h].wait_recv()
                    if h < N_DEV - 2:
                        ag[(d["sgn"], j)].append(ag_send(d, j, h + 1))
                    out_ref[pl.ds(r_idx * CHUNK, CHUNK), cj:cj + QCOL] = (
                        d["ag"][r_idx, :, j * QCOL:(j + 1) * QCOL].astype(
                            jnp.float32)
                    )
        for key, rd in ag.items():
            for rdma in rd:
                rdma.wait_send()

    return pl.pallas_call(
        body,
        out_shape=jax.ShapeDtypeStruct((M, N), jnp.float32),
        in_specs=[pl.BlockSpec(memory_space=pltpu.VMEM)],
        out_specs=pl.BlockSpec(memory_space=pltpu.VMEM),
        scratch_shapes=[
            pltpu.VMEM((2, CHUNK, HALF), jnp.bfloat16),
            pltpu.VMEM((2, CHUNK, HALF), jnp.bfloat16),
            pltpu.VMEM((N_DEV, CHUNK, HALF), jnp.bfloat16),
            pltpu.VMEM((N_DEV, CHUNK, HALF), jnp.bfloat16),
            pltpu.SemaphoreType.DMA((N_DEV - 1, SUB)),
            pltpu.SemaphoreType.DMA((N_DEV - 1, SUB)),
            pltpu.SemaphoreType.DMA((N_DEV - 1, SUB)),
            pltpu.SemaphoreType.DMA((N_DEV - 1, SUB)),
        ],
        compiler_params=pltpu.CompilerParams(collective_id=0),
    )(partial)


# device time: 143165 ns/iter; 2.9812x vs baseline; 1.0298x over previous
import jax
import jax.numpy as jnp
from jax import lax
from jax.experimental import pallas as pl
from jax.experimental.pallas import tpu as pltpu

N_DEV = 16
M = 2048
N = 2048
CHUNK = M // N_DEV
HALF = N // 2
SUB = 4
QCOL = HALF // SUB


def kernel(A, B):
    partial = jnp.dot(
        A.astype(jnp.bfloat16),
        B.astype(jnp.bfloat16),
        preferred_element_type=jnp.float32,
    )

    def body(p_ref, out_ref, stage_r, stage_l, ag_r, ag_l,
             send_sems_r, recv_sems_r, send_sems_l, recv_sems_l):
        my = lax.axis_index("i")
        left = lax.rem(my - 1 + N_DEV, N_DEV)
        right = lax.rem(my + 1, N_DEV)

        barrier_sem = pltpu.get_barrier_semaphore()
        for nbr in (left, right):
            pl.semaphore_signal(
                barrier_sem, inc=1,
                device_id=(nbr,), device_id_type=pl.DeviceIdType.MESH,
            )
        pl.semaphore_wait(barrier_sem, 2)

        dirs = [
            dict(sgn=-1, nbr=right, col=0, stage=stage_r, ag=ag_r,
                 ssem=send_sems_r, rsem=recv_sems_r),
            dict(sgn=+1, nbr=left, col=HALF, stage=stage_l, ag=ag_l,
                 ssem=send_sems_l, rsem=recv_sems_l),
        ]

        def idx(k):
            return lax.rem(k + 4 * N_DEV, N_DEV)

        def send(d, j, h, src_buf, src_slice):
            jsl = slice(j * QCOL, (j + 1) * QCOL)
            rdma = pltpu.make_async_remote_copy(
                src_ref=src_buf.at[src_slice, :, jsl],
                dst_ref=d["ag"].at[idx(my + d["sgn"] * h), :, jsl],
                send_sem=d["ssem"].at[h, j],
                recv_sem=d["rsem"].at[h, j],
                device_id=(d["nbr"],),
                device_id_type=pl.DeviceIdType.MESH,
            )
            rdma.start()
            return rdma

        rs = {}
        for d in dirs:
            s0 = idx(my)
            d["stage"][0] = p_ref[
                pl.ds(s0 * CHUNK, CHUNK), d["col"]:d["col"] + HALF
            ].astype(jnp.bfloat16)
            for j in range(SUB):
                rs[(d["sgn"], j)] = [send(d, j, 0, d["stage"], 0)]

        for h in range(N_DEV - 1):
            for j in range(SUB):
                for d in dirs:
                    cj = d["col"] + j * QCOL
                    r_idx = idx(my + d["sgn"] * (h + 1))
                    rd = rs[(d["sgn"], j)]
                    rd[h].wait_recv()
                    out_ref[pl.ds(r_idx * CHUNK, CHUNK), cj:cj + QCOL] = (
                        p_ref[pl.ds(r_idx * CHUNK, CHUNK), cj:cj + QCOL]
                        + d["ag"][r_idx, :, j * QCOL:(j + 1) * QCOL].astype(
                            jnp.float32)
                    )
                    if h < N_DEV - 2:
                        slot = (h + 1) % 2
                        if h >= 1:
                            rd[h - 1].wait_send()
                        d["stage"][slot, :, j * QCOL:(j + 1) * QCOL] = out_ref[
                            pl.ds(r_idx * CHUNK, CHUNK), cj:cj + QCOL
                        ].astype(jnp.bfloat16)
                        rd.append(send(d, j, h + 1, d["stage"], slot))
        for key, rd in rs.items():
            rd[N_DEV - 3].wait_send()
            rd[N_DEV - 2].wait_send()

        def ag_send(d, j, h):
            jsl = slice(j * QCOL, (j + 1) * QCOL)
            s_idx = idx(my - d["sgn"] + d["sgn"] * h)
            rdma = pltpu.make_async_remote_copy(
                src_ref=d["ag"].at[s_idx, :, jsl],
                dst_ref=d["ag"].at[s_idx, :, jsl],
                send_sem=d["ssem"].at[h, j],
                recv_sem=d["rsem"].at[h, j],
                device_id=(d["nbr"],),
                device_id_type=pl.DeviceIdType.MESH,
            )
            rdma.start()
            return rdma

        ag = {}
        for d in dirs:
            own = idx(my - d["sgn"])
            d["ag"][own] = out_ref[
                pl.ds(own * CHUNK, CHUNK), d["col"]:d["col"] + HALF
            ].astype(jnp.bfloat16)
            for j in range(SUB):
                ag[(d["sgn"], j)] = [ag_send(d, j, 0)]

        for h in range(N_DEV - 1):
            for j in range(SUB):
                for d in dirs:
                    cj = d["col"] + j * QCOL
                    r_idx = idx(my + d["sgn"] * h)
                    ag[(d["sgn"], j)][h].wait_recv()
                    if h < N_DEV - 2:
                        ag[(d["sgn"], j)].append(ag_send(d, j, h + 1))
                    out_ref[pl.ds(r_idx * CHUNK, CHUNK), cj:cj + QCOL] = (
                        d["ag"][r_idx, :, j * QCOL:(j + 1) * QCOL].astype(
                            jnp.float32)
                    )
        for key, rd in ag.items():
            for rdma in rd:
                rdma.wait_send()

    return pl.pallas_call(
        body,
        out_shape=jax.ShapeDtypeStruct((M, N), jnp.float32),
        in_specs=[pl.BlockSpec(memory_space=pltpu.VMEM)],
        out_specs=pl.BlockSpec(memory_space=pltpu.VMEM),
        scratch_shapes=[
            pltpu.VMEM((2, CHUNK, HALF), jnp.bfloat16),
            pltpu.VMEM((2, CHUNK, HALF), jnp.bfloat16),
            pltpu.VMEM((N_DEV, CHUNK, HALF), jnp.bfloat16),
            pltpu.VMEM((N_DEV, CHUNK, HALF), jnp.bfloat16),
            pltpu.SemaphoreType.DMA((N_DEV - 1, SUB)),
            pltpu.SemaphoreType.DMA((N_DEV - 1, SUB)),
            pltpu.SemaphoreType.DMA((N_DEV - 1, SUB)),
            pltpu.SemaphoreType.DMA((N_DEV - 1, SUB)),
        ],
        compiler_params=pltpu.CompilerParams(collective_id=0),
    )(partial)


# device time: 142830 ns/iter; 2.9881x vs baseline; 1.0023x over previous
import jax
import jax.numpy as jnp
from jax import lax
from jax.experimental import pallas as pl
from jax.experimental.pallas import tpu as pltpu

N_DEV = 16
M = 2048
N = 2048
CHUNK = M // N_DEV
HALF = N // 2
SUB = 4
QCOL = HALF // SUB


def kernel(A, B):
    partial = jnp.dot(
        A.astype(jnp.bfloat16),
        B.astype(jnp.bfloat16),
        preferred_element_type=jnp.float32,
    )

    def body(p_ref, out_ref, stage_r, stage_l, ag_r, ag_l,
             send_sems_r, recv_sems_r, send_sems_l, recv_sems_l):
        my = lax.axis_index("i")
        left = lax.rem(my - 1 + N_DEV, N_DEV)
        right = lax.rem(my + 1, N_DEV)

        barrier_sem = pltpu.get_barrier_semaphore()
        for nbr in (left, right):
            pl.semaphore_signal(
                barrier_sem, inc=1,
                device_id=(nbr,), device_id_type=pl.DeviceIdType.MESH,
            )
        pl.semaphore_wait(barrier_sem, 2)

        dirs = [
            dict(sgn=-1, nbr=right, col=0, stage=stage_r, ag=ag_r,
                 ssem=send_sems_r, rsem=recv_sems_r),
            dict(sgn=+1, nbr=left, col=HALF, stage=stage_l, ag=ag_l,
                 ssem=send_sems_l, rsem=recv_sems_l),
        ]

        def idx(k):
            return lax.rem(k + 4 * N_DEV, N_DEV)

        def send(d, j, h, src_buf, src_slice):
            jsl = slice(j * QCOL, (j + 1) * QCOL)
            rdma = pltpu.make_async_remote_copy(
                src_ref=src_buf.at[src_slice, :, jsl],
                dst_ref=d["ag"].at[idx(my + d["sgn"] * h), :, jsl],
                send_sem=d["ssem"].at[h, j],
                recv_sem=d["rsem"].at[h, j],
                device_id=(d["nbr"],),
                device_id_type=pl.DeviceIdType.MESH,
            )
            rdma.start()
            return rdma

        rs = {}
        for d in dirs:
            s0 = idx(my)
            d["stage"][0] = p_ref[
                pl.ds(s0 * CHUNK, CHUNK), d["col"]:d["col"] + HALF
            ].astype(jnp.bfloat16)
            for j in range(SUB):
                rs[(d["sgn"], j)] = [send(d, j, 0, d["stage"], 0)]

        for h in range(N_DEV - 1):
            for j in range(SUB):
                for d in dirs:
                    cj = d["col"] + j * QCOL
                    r_idx = idx(my + d["sgn"] * (h + 1))
                    rd = rs[(d["sgn"], j)]
                    rd[h].wait_recv()
                    tmp = (
                        p_ref[pl.ds(r_idx * CHUNK, CHUNK), cj:cj + QCOL]
                        + d["ag"][r_idx, :, j * QCOL:(j + 1) * QCOL].astype(
                            jnp.float32)
                    )
                    if h < N_DEV - 2:
                        slot = (h + 1) % 2
                        if h >= 1:
                            rd[h - 1].wait_send()
                        d["stage"][slot, :, j * QCOL:(j + 1) * QCOL] = (
                            tmp.astype(jnp.bfloat16))
                        rd.append(send(d, j, h + 1, d["stage"], slot))
                    out_ref[pl.ds(r_idx * CHUNK, CHUNK), cj:cj + QCOL] = tmp
        for key, rd in rs.items():
            rd[N_DEV - 3].wait_send()
            rd[N_DEV - 2].wait_send()

        def ag_send(d, j, h):
            jsl = slice(j * QCOL, (j + 1) * QCOL)
            s_idx = idx(my - d["sgn"] + d["sgn"] * h)
            rdma = pltpu.make_async_remote_copy(
                src_ref=d["ag"].at[s_idx, :, jsl],
                dst_ref=d["ag"].at[s_idx, :, jsl],
                send_sem=d["ssem"].at[h, j],
                recv_sem=d["rsem"].at[h, j],
                device_id=(d["nbr"],),
                device_id_type=pl.DeviceIdType.MESH,
            )
            rdma.start()
            return rdma

        ag = {}
        for d in dirs:
            own = idx(my - d["sgn"])
            d["ag"][own] = out_ref[
                pl.ds(own * CHUNK, CHUNK), d["col"]:d["col"] + HALF
            ].astype(jnp.bfloat16)
            for j in range(SUB):
                ag[(d["sgn"], j)] = [ag_send(d, j, 0)]

        for h in range(N_DEV - 1):
            for j in range(SUB):
                for d in dirs:
                    cj = d["col"] + j * QCOL
                    r_idx = idx(my + d["sgn"] * h)
                    ag[(d["sgn"], j)][h].wait_recv()
                    if h < N_DEV - 2:
                        ag[(d["sgn"], j)].append(ag_send(d, j, h + 1))
                    out_ref[pl.ds(r_idx * CHUNK, CHUNK), cj:cj + QCOL] = (
                        d["ag"][r_idx, :, j * QCOL:(j + 1) * QCOL].astype(
                            jnp.float32)
                    )
        for key, rd in ag.items():
            for rdma in rd:
                rdma.wait_send()

    return pl.pallas_call(
        body,
        out_shape=jax.ShapeDtypeStruct((M, N), jnp.float32),
        in_specs=[pl.BlockSpec(memory_space=pltpu.VMEM)],
        out_specs=pl.BlockSpec(memory_space=pltpu.VMEM),
        scratch_shapes=[
            pltpu.VMEM((2, CHUNK, HALF), jnp.bfloat16),
            pltpu.VMEM((2, CHUNK, HALF), jnp.bfloat16),
            pltpu.VMEM((N_DEV, CHUNK, HALF), jnp.bfloat16),
            pltpu.VMEM((N_DEV, CHUNK, HALF), jnp.bfloat16),
            pltpu.SemaphoreType.DMA((N_DEV - 1, SUB)),
            pltpu.SemaphoreType.DMA((N_DEV - 1, SUB)),
            pltpu.SemaphoreType.DMA((N_DEV - 1, SUB)),
            pltpu.SemaphoreType.DMA((N_DEV - 1, SUB)),
        ],
        compiler_params=pltpu.CompilerParams(collective_id=0),
    )(partial)
